# Initial kernel scaffold; baseline (speedup 1.0000x reference)
import jax
import jax.numpy as jnp
from jax.experimental import pallas as pl


def kernel(query_feature, key_feature, key_ind, W, bias, gamma, beta):
    raise NotImplementedError("write your pallas kernel here")


# speedup vs baseline: 5.3352x; 5.3352x over previous
import functools

import jax
import jax.numpy as jnp
from jax import lax
from jax.experimental import pallas as pl
from jax.experimental.pallas import tpu as pltpu
from jax.experimental.pallas import tpu_sc as plsc

N = 10000
C = 128
K = 32
OUT_C = 64

NPAD = 10240
NW = 32
QPW = NPAD // NW
QPC = 4
NCHUNK = QPW // QPC
IPC = QPC * K
TBL_W = 128
NKF = float(N * K)


def _prep_body(kft_ref, qft_ref, wt_ref, bias_ref, at_ref, bqt_ref):
    w1t = wt_ref[0:C, :]
    dwt = wt_ref[C : 2 * C, :] - w1t
    at = jnp.dot(kft_ref[...], w1t, preferred_element_type=jnp.float32)
    at_ref[...] = jnp.pad(at, ((0, 0), (0, TBL_W - OUT_C)))
    bqt_ref[...] = (
        jnp.dot(qft_ref[...], dwt, preferred_element_type=jnp.float32)
        + bias_ref[...]
    )


def _prep(kft, qft, wt, bias2):
    return pl.pallas_call(
        _prep_body,
        out_shape=[
            jax.ShapeDtypeStruct((NPAD, TBL_W), jnp.float32),
            jax.ShapeDtypeStruct((NPAD, OUT_C), jnp.float32),
        ],
    )(kft, qft, wt, bias2)


def _sc_body(at_hbm, idx_hbm, out_hbm, idx_v, rows0, rows1, res_v, sem0, sem1):
    wid = lax.axis_index("s") * 2 + lax.axis_index("c")
    pltpu.sync_copy(idx_hbm.at[wid], idx_v)

    rows = (rows0, rows1)
    sems = (sem0, sem1)

    pltpu.async_copy(at_hbm.at[idx_v.at[0]], rows0, sem0)
    pltpu.async_copy(at_hbm.at[idx_v.at[1]], rows1, sem1)

    neg_inf = jnp.full((16,), -jnp.inf, jnp.float32)
    pos_inf = jnp.full((16,), jnp.inf, jnp.float32)
    zeros = jnp.zeros((16,), jnp.float32)

    def compute_chunk(g, rbuf):
        for q in range(QPC):
            base = q * K

            def kbody(k, carry):
                out = []
                for cg in range(4):
                    s_a, q_a, mx_a, mn_a = carry[cg]
                    v = rbuf[base + k, pl.ds(cg * 16, 16)]
                    out.append(
                        (
                            s_a + v,
                            q_a + v * v,
                            jnp.maximum(mx_a, v),
                            jnp.minimum(mn_a, v),
                        )
                    )
                return tuple(out)

            init = tuple((zeros, zeros, neg_inf, pos_inf) for _ in range(4))
            acc = lax.fori_loop(0, K, kbody, init, unroll=4)
            qrow = g * QPC + q
            for cg in range(4):
                s_a, q_a, mx_a, mn_a = acc[cg]
                res_v[qrow, pl.ds(cg * 16, 16)] = s_a
                res_v[qrow, pl.ds(64 + cg * 16, 16)] = q_a
                res_v[qrow, pl.ds(128 + cg * 16, 16)] = mx_a
                res_v[qrow, pl.ds(192 + cg * 16, 16)] = mn_a

    def outer(i, _):
        for b in range(2):
            g = 2 * i + b
            pltpu.make_async_copy(at_hbm.at[idx_v.at[g]], rows[b], sems[b]).wait()
            compute_chunk(g, rows[b])

            @pl.when(g + 2 < NCHUNK)
            def _():
                pltpu.async_copy(at_hbm.at[idx_v.at[g + 2]], rows[b], sems[b])

        return 0

    lax.fori_loop(0, NCHUNK // 2, outer, 0)
    pltpu.sync_copy(res_v, out_hbm.at[pl.ds(wid * QPW, QPW)])


def _sc_gather(at, idx3):
    mesh = plsc.VectorSubcoreMesh(
        core_axis_name="c", subcore_axis_name="s", num_cores=2, num_subcores=16
    )
    fn = pl.kernel(
        _sc_body,
        out_type=jax.ShapeDtypeStruct((NPAD, 4 * OUT_C), jnp.float32),
        mesh=mesh,
        scratch_types=[
            pltpu.VMEM((NCHUNK, IPC), jnp.int32),
            pltpu.VMEM((IPC, TBL_W), jnp.float32),
            pltpu.VMEM((IPC, TBL_W), jnp.float32),
            pltpu.VMEM((QPW, 4 * OUT_C), jnp.float32),
            pltpu.SemaphoreType.DMA,
            pltpu.SemaphoreType.DMA,
        ],
    )
    return fn(at, idx3)


def _stats_body(r_ref, bqt_ref, gamma_ref, beta_ref, out_ref):
    valid = (
        lax.broadcasted_iota(jnp.int32, (NPAD, 1), 0) < N
    ).astype(jnp.float32)
    s_g = r_ref[:, 0:OUT_C] * valid
    q_g = r_ref[:, OUT_C : 2 * OUT_C] * valid
    b_g = bqt_ref[...] * valid

    sum_s = jnp.sum(s_g, axis=0, keepdims=True)
    sum_q = jnp.sum(q_g, axis=0, keepdims=True)
    cross = jnp.sum(s_g * b_g, axis=0, keepdims=True)
    sum_b = jnp.sum(b_g, axis=0, keepdims=True)
    sum_b2 = jnp.sum(b_g * b_g, axis=0, keepdims=True)

    mean = (sum_s + K * sum_b) * (1.0 / NKF)
    ey2 = (sum_q + 2.0 * cross + K * sum_b2) * (1.0 / NKF)
    var = ey2 - mean * mean
    scale = gamma_ref[...] * lax.rsqrt(var + 1e-5)
    shift = beta_ref[...] - scale * mean
    out_ref[...] = jnp.concatenate([scale, shift], axis=0)


def _stats(r, bqt, gamma2, beta2):
    return pl.pallas_call(
        _stats_body,
        out_shape=jax.ShapeDtypeStruct((2, OUT_C), jnp.float32),
    )(r, bqt, gamma2, beta2)


def _apply_body(r_ref, bqt_ref, st_ref, out_ref):
    scale = st_ref[0:1, :]
    shift = st_ref[1:2, :]
    bq = bqt_ref[...]
    mx = r_ref[:, 2 * OUT_C : 3 * OUT_C] + bq
    mn = r_ref[:, 3 * OUT_C : 4 * OUT_C] + bq
    m = jnp.where(scale >= 0.0, mx, mn)
    out_ref[...] = jnp.maximum(m * scale + shift, 0.0)


def _apply(r, bqt, st):
    blk = 1024
    grid = NPAD // blk
    return pl.pallas_call(
        _apply_body,
        grid=(grid,),
        in_specs=[
            pl.BlockSpec((blk, 4 * OUT_C), lambda i: (i, 0)),
            pl.BlockSpec((blk, OUT_C), lambda i: (i, 0)),
            pl.BlockSpec((2, OUT_C), lambda i: (0, 0)),
        ],
        out_specs=pl.BlockSpec((blk, OUT_C), lambda i: (i, 0)),
        out_shape=jax.ShapeDtypeStruct((NPAD, OUT_C), jnp.float32),
    )(r, bqt, st)


def kernel(query_feature, key_feature, key_ind, W, bias, gamma, beta):
    kf = key_feature[0]
    qf = query_feature[0]
    kft = jnp.pad(kf, ((0, 0), (0, NPAD - N))).T
    qft = jnp.pad(qf, ((0, 0), (0, NPAD - N))).T
    wt = W.T
    bias2 = bias.reshape(1, OUT_C)
    gamma2 = gamma.reshape(1, OUT_C)
    beta2 = beta.reshape(1, OUT_C)

    at, bqt = _prep(kft, qft, wt, bias2)

    idx = jnp.pad(key_ind[0].astype(jnp.int32), ((0, NPAD - N), (0, 0)))
    idx3 = idx.reshape(NW, NCHUNK, IPC)
    r = _sc_gather(at, idx3)

    st = _stats(r, bqt, gamma2, beta2)
    out_t = _apply(r, bqt, st)
    return out_t[:N].T[None]


# speedup vs baseline: 5.3737x; 1.0072x over previous
import functools

import jax
import jax.numpy as jnp
from jax import lax
from jax.experimental import pallas as pl
from jax.experimental.pallas import tpu as pltpu
from jax.experimental.pallas import tpu_sc as plsc

N = 10000
C = 128
K = 32
OUT_C = 64

NPAD = 10240
NW = 32
QPW = NPAD // NW
QPC = 4
NCHUNK = QPW // QPC
IPC = QPC * K
TBL_W = 128
NKF = float(N * K)


def _prep_body(kft_ref, qft_ref, wt_ref, bias_ref, at_ref, bqt_ref):
    w1t = wt_ref[0:C, :]
    dwt = wt_ref[C : 2 * C, :] - w1t
    at = jnp.dot(kft_ref[...], w1t, preferred_element_type=jnp.float32)
    at_ref[...] = jnp.pad(at, ((0, 0), (0, TBL_W - OUT_C)))
    bqt_ref[...] = (
        jnp.dot(qft_ref[...], dwt, preferred_element_type=jnp.float32)
        + bias_ref[...]
    )


def _prep(kft, qft, wt, bias2):
    return pl.pallas_call(
        _prep_body,
        out_shape=[
            jax.ShapeDtypeStruct((NPAD, TBL_W), jnp.float32),
            jax.ShapeDtypeStruct((NPAD, OUT_C), jnp.float32),
        ],
    )(kft, qft, wt, bias2)


def _sc_body(at_hbm, idx_hbm, out_hbm, idx_v, rows0, rows1, res_v, sem0, sem1):
    wid = lax.axis_index("s") * 2 + lax.axis_index("c")
    pltpu.sync_copy(idx_hbm.at[wid], idx_v)

    rows = (rows0, rows1)
    sems = (sem0, sem1)

    pltpu.async_copy(at_hbm.at[idx_v.at[0]], rows0, sem0)
    pltpu.async_copy(at_hbm.at[idx_v.at[1]], rows1, sem1)

    def compute_chunk(g, rbuf):
        for q in range(QPC):
            base = q * K
            acc = None
            for k in range(K):
                vs = [rbuf[base + k, pl.ds(cg * 16, 16)] for cg in range(4)]
                if acc is None:
                    acc = [(v, v * v, v, v) for v in vs]
                else:
                    acc = [
                        (
                            s_a + v,
                            q_a + v * v,
                            jnp.maximum(mx_a, v),
                            jnp.minimum(mn_a, v),
                        )
                        for (s_a, q_a, mx_a, mn_a), v in zip(acc, vs)
                    ]
            qrow = g * QPC + q
            for cg in range(4):
                s_a, q_a, mx_a, mn_a = acc[cg]
                res_v[qrow, pl.ds(cg * 16, 16)] = s_a
                res_v[qrow, pl.ds(64 + cg * 16, 16)] = q_a
                res_v[qrow, pl.ds(128 + cg * 16, 16)] = mx_a
                res_v[qrow, pl.ds(192 + cg * 16, 16)] = mn_a

    def outer(i, _):
        for b in range(2):
            g = 2 * i + b
            pltpu.make_async_copy(at_hbm.at[idx_v.at[g]], rows[b], sems[b]).wait()
            compute_chunk(g, rows[b])

            @pl.when(g + 2 < NCHUNK)
            def _():
                pltpu.async_copy(at_hbm.at[idx_v.at[g + 2]], rows[b], sems[b])

        return 0

    lax.fori_loop(0, NCHUNK // 2, outer, 0)
    pltpu.sync_copy(res_v, out_hbm.at[pl.ds(wid * QPW, QPW)])


def _sc_gather(at, idx3):
    mesh = plsc.VectorSubcoreMesh(
        core_axis_name="c", subcore_axis_name="s", num_cores=2, num_subcores=16
    )
    fn = pl.kernel(
        _sc_body,
        out_type=jax.ShapeDtypeStruct((NPAD, 4 * OUT_C), jnp.float32),
        mesh=mesh,
        scratch_types=[
            pltpu.VMEM((NCHUNK, IPC), jnp.int32),
            pltpu.VMEM((IPC, TBL_W), jnp.float32),
            pltpu.VMEM((IPC, TBL_W), jnp.float32),
            pltpu.VMEM((QPW, 4 * OUT_C), jnp.float32),
            pltpu.SemaphoreType.DMA,
            pltpu.SemaphoreType.DMA,
        ],
    )
    return fn(at, idx3)


def _stats_body(r_ref, bqt_ref, gamma_ref, beta_ref, out_ref):
    valid = (
        lax.broadcasted_iota(jnp.int32, (NPAD, 1), 0) < N
    ).astype(jnp.float32)
    s_g = r_ref[:, 0:OUT_C] * valid
    q_g = r_ref[:, OUT_C : 2 * OUT_C] * valid
    b_g = bqt_ref[...] * valid

    sum_s = jnp.sum(s_g, axis=0, keepdims=True)
    sum_q = jnp.sum(q_g, axis=0, keepdims=True)
    cross = jnp.sum(s_g * b_g, axis=0, keepdims=True)
    sum_b = jnp.sum(b_g, axis=0, keepdims=True)
    sum_b2 = jnp.sum(b_g * b_g, axis=0, keepdims=True)

    mean = (sum_s + K * sum_b) * (1.0 / NKF)
    ey2 = (sum_q + 2.0 * cross + K * sum_b2) * (1.0 / NKF)
    var = ey2 - mean * mean
    scale = gamma_ref[...] * lax.rsqrt(var + 1e-5)
    shift = beta_ref[...] - scale * mean
    out_ref[...] = jnp.concatenate([scale, shift], axis=0)


def _stats(r, bqt, gamma2, beta2):
    return pl.pallas_call(
        _stats_body,
        out_shape=jax.ShapeDtypeStruct((2, OUT_C), jnp.float32),
    )(r, bqt, gamma2, beta2)


def _apply_body(r_ref, bqt_ref, st_ref, out_ref):
    scale = st_ref[0:1, :]
    shift = st_ref[1:2, :]
    bq = bqt_ref[...]
    mx = r_ref[:, 2 * OUT_C : 3 * OUT_C] + bq
    mn = r_ref[:, 3 * OUT_C : 4 * OUT_C] + bq
    m = jnp.where(scale >= 0.0, mx, mn)
    out_ref[...] = jnp.maximum(m * scale + shift, 0.0)


def _apply(r, bqt, st):
    blk = 1024
    grid = NPAD // blk
    return pl.pallas_call(
        _apply_body,
        grid=(grid,),
        in_specs=[
            pl.BlockSpec((blk, 4 * OUT_C), lambda i: (i, 0)),
            pl.BlockSpec((blk, OUT_C), lambda i: (i, 0)),
            pl.BlockSpec((2, OUT_C), lambda i: (0, 0)),
        ],
        out_specs=pl.BlockSpec((blk, OUT_C), lambda i: (i, 0)),
        out_shape=jax.ShapeDtypeStruct((NPAD, OUT_C), jnp.float32),
    )(r, bqt, st)


def kernel(query_feature, key_feature, key_ind, W, bias, gamma, beta):
    kf = key_feature[0]
    qf = query_feature[0]
    kft = jnp.pad(kf, ((0, 0), (0, NPAD - N))).T
    qft = jnp.pad(qf, ((0, 0), (0, NPAD - N))).T
    wt = W.T
    bias2 = bias.reshape(1, OUT_C)
    gamma2 = gamma.reshape(1, OUT_C)
    beta2 = beta.reshape(1, OUT_C)

    at, bqt = _prep(kft, qft, wt, bias2)

    idx = jnp.pad(key_ind[0].astype(jnp.int32), ((0, NPAD - N), (0, 0)))
    idx3 = idx.reshape(NW, NCHUNK, IPC)
    r = _sc_gather(at, idx3)

    st = _stats(r, bqt, gamma2, beta2)
    out_t = _apply(r, bqt, st)
    return out_t[:N].T[None]
